# baseline (speedup 1.0000x reference)
import jax
import jax.numpy as jnp
from jax.experimental import pallas as pl
from jax.experimental.pallas import tpu as pltpu


_BLOCK_ROWS = 16384


def kernel(inputs):
    n, c = inputs.shape
    block_rows = _BLOCK_ROWS if n % _BLOCK_ROWS == 0 else n
    grid = (n // block_rows,)

    def _copy_body(x_hbm, y_ref, ld_ref, sem):
        i = pl.program_id(0)
        pltpu.make_async_copy(
            x_hbm.at[pl.ds(i * block_rows, block_rows)], y_ref, sem
        ).start()
        ld_ref[...] = jnp.zeros_like(ld_ref)
        pltpu.make_async_copy(
            x_hbm.at[pl.ds(i * block_rows, block_rows)], y_ref, sem
        ).wait()

    outputs, ld = pl.pallas_call(
        _copy_body,
        grid=grid,
        in_specs=[pl.BlockSpec(memory_space=pl.ANY)],
        out_specs=[
            pl.BlockSpec((block_rows, c), lambda i: (i, 0)),
            pl.BlockSpec((block_rows,), lambda i: (i,)),
        ],
        out_shape=[
            jax.ShapeDtypeStruct((n, c), inputs.dtype),
            jax.ShapeDtypeStruct((n,), jnp.float32),
        ],
        scratch_shapes=[pltpu.SemaphoreType.DMA],
        compiler_params=pltpu.CompilerParams(
            dimension_semantics=("arbitrary",),
            vmem_limit_bytes=128 * 1024 * 1024,
        ),
    )(inputs)
    return (outputs, ld)


# speedup vs baseline: 1.1166x; 1.1166x over previous
import jax
import jax.numpy as jnp
from jax.experimental import pallas as pl
from jax.experimental.pallas import tpu as pltpu


_BLOCK_ROWS = 16128


def _copy_body(x_ref, y_ref, ld_ref):
    y_ref[...] = x_ref[...]
    ld_ref[...] = jnp.zeros_like(ld_ref)


def kernel(inputs):
    n, c = inputs.shape
    block_rows = min(_BLOCK_ROWS, n)
    grid = (pl.cdiv(n, block_rows),)
    ld_block = 1024 * pl.cdiv(n, 1024 * grid[0])
    outputs, ld = pl.pallas_call(
        _copy_body,
        grid=grid,
        in_specs=[pl.BlockSpec((block_rows, c), lambda i: (i, 0))],
        out_specs=[
            pl.BlockSpec((block_rows, c), lambda i: (i, 0)),
            pl.BlockSpec((ld_block,), lambda i: (i,)),
        ],
        out_shape=[
            jax.ShapeDtypeStruct((n, c), inputs.dtype),
            jax.ShapeDtypeStruct((n,), jnp.float32),
        ],
        compiler_params=pltpu.CompilerParams(
            dimension_semantics=("parallel",),
            vmem_limit_bytes=128 * 1024 * 1024,
        ),
    )(inputs)
    return (outputs, ld)
